# Initial kernel scaffold; baseline (speedup 1.0000x reference)
import jax
import jax.numpy as jnp
from jax.experimental import pallas as pl


def kernel(x, edge_index, W1, b1, W2, b2, Wc, bc):
    raise NotImplementedError("write your pallas kernel here")


# speedup vs baseline: 13.6948x; 13.6948x over previous
import functools

import jax
import jax.numpy as jnp
from jax import lax
from jax.experimental import pallas as pl
from jax.experimental.pallas import tpu as pltpu
from jax.experimental.pallas import tpu_sc as plsc

N = 10000
NP = 10240
DI = 128
DH = 64
E = 320000
NWORK = 32
CHUNK = 128
NCH = 80
EPW = NCH * CHUNK
EP = NWORK * EPW
SLAB = NP // 16

@functools.cache
def _sc_kernels():
    mesh = plsc.VectorSubcoreMesh(
        core_axis_name="c", subcore_axis_name="s", num_cores=2, num_subcores=16)

    @functools.partial(
        pl.kernel,
        out_type=jax.ShapeDtypeStruct((2, NP, 16), jnp.float32),
        mesh=mesh,
        compiler_params=pltpu.CompilerParams(use_tc_tiling_on_sc=False),
        scratch_types=[
            pltpu.VMEM((NCH, CHUNK), jnp.int32),
            pltpu.VMEM((CHUNK, 16), jnp.float32),
            pltpu.VMEM_SHARED((NP, 16), jnp.float32),
        ],
    )
    def deg_kernel(dst_hbm, ones_hbm, zeros_hbm, out_hbm, dst_v, ones_v, acc):
        cid = lax.axis_index("c")
        sid = lax.axis_index("s")
        wid = cid * 16 + sid
        slab = pl.ds(sid * SLAB, SLAB)
        pltpu.sync_copy(dst_hbm.at[wid], dst_v)
        pltpu.sync_copy(ones_hbm, ones_v)
        pltpu.sync_copy(zeros_hbm.at[slab], acc.at[slab])
        plsc.subcore_barrier()

        def body(j, carry):
            pltpu.sync_copy(ones_v, acc.at[dst_v.at[j]], add=True)
            return carry

        lax.fori_loop(0, NCH, body, 0)
        plsc.subcore_barrier()
        pltpu.sync_copy(acc.at[slab], out_hbm.at[cid].at[slab])

    @functools.partial(
        pl.kernel,
        out_type=jax.ShapeDtypeStruct((2, NP, DH), jnp.float32),
        mesh=mesh,
        compiler_params=pltpu.CompilerParams(use_tc_tiling_on_sc=False),
        scratch_types=[
            pltpu.VMEM((NCH, CHUNK), jnp.int32),
            pltpu.VMEM((NCH, CHUNK), jnp.int32),
            pltpu.VMEM((CHUNK, DH), jnp.float32),
            pltpu.VMEM_SHARED((NP, DH), jnp.float32),
            pltpu.SemaphoreType.DMA,
        ],
    )
    def agg_kernel(g_hbm, src_hbm, dst_hbm, zeros_hbm, out_hbm,
                   src_v, dst_v, rows, acc, gsem):
        cid = lax.axis_index("c")
        sid = lax.axis_index("s")
        wid = cid * 16 + sid
        slab = pl.ds(sid * SLAB, SLAB)
        pltpu.sync_copy(src_hbm.at[wid], src_v)
        pltpu.sync_copy(dst_hbm.at[wid], dst_v)
        pltpu.sync_copy(zeros_hbm.at[slab], acc.at[slab])
        plsc.subcore_barrier()

        def body(j, carry):
            pltpu.async_copy(g_hbm.at[src_v.at[j]], rows, gsem).wait()
            pltpu.sync_copy(rows, acc.at[dst_v.at[j]], add=True)
            return carry

        lax.fori_loop(0, NCH, body, 0)
        plsc.subcore_barrier()
        pltpu.sync_copy(acc.at[slab], out_hbm.at[cid].at[slab])

    return deg_kernel, agg_kernel


RB = 512
GRID = NP // RB


def _dinv_from(deg_ref):
    d = deg_ref[0, :, 0:1] + deg_ref[1, :, 0:1] + 1.0
    return lax.rsqrt(d)


def _stage_a_body(x_ref, w_ref, deg_ref, g_ref):
    dinv = _dinv_from(deg_ref)
    h = jnp.dot(x_ref[...], w_ref[...], preferred_element_type=jnp.float32)
    g_ref[...] = h * dinv


def _stage_b_body(a_ref, g1_ref, deg_ref, w_ref, b_ref, g2_ref):
    dinv = _dinv_from(deg_ref)
    agg = a_ref[0] + a_ref[1] + g1_ref[...]
    z = jnp.maximum(agg * dinv + b_ref[...], 0.0)
    h2 = jnp.dot(z, w_ref[...], preferred_element_type=jnp.float32)
    g2_ref[...] = h2 * dinv


def _stage_c_body(a_ref, g2_ref, deg_ref, w_ref, b2_ref, bc_ref, out_ref):
    dinv = _dinv_from(deg_ref)
    agg = a_ref[0] + a_ref[1] + g2_ref[...]
    z = agg * dinv + b2_ref[...]
    out_ref[...] = (
        jnp.dot(z, w_ref[...], preferred_element_type=jnp.float32)
        + bc_ref[...])


def _row_spec(d):
    return pl.BlockSpec((RB, d), lambda i: (i, 0))


_deg_spec = pl.BlockSpec((2, RB, 16), lambda i: (0, i, 0))
_part_spec = pl.BlockSpec((2, RB, DH), lambda i: (0, i, 0))


def _full_spec(shape):
    return pl.BlockSpec(shape, lambda i: tuple(0 for _ in shape))


_stage_a = pl.pallas_call(
    _stage_a_body,
    out_shape=jax.ShapeDtypeStruct((NP, DH), jnp.float32),
    grid=(GRID,),
    in_specs=[_row_spec(DI), _full_spec((DI, DH)), _deg_spec],
    out_specs=_row_spec(DH),
)

_stage_b = pl.pallas_call(
    _stage_b_body,
    out_shape=jax.ShapeDtypeStruct((NP, DH), jnp.float32),
    grid=(GRID,),
    in_specs=[_part_spec, _row_spec(DH), _deg_spec,
              _full_spec((DH, DH)), _full_spec((1, DH))],
    out_specs=_row_spec(DH),
)

_stage_c = pl.pallas_call(
    _stage_c_body,
    out_shape=jax.ShapeDtypeStruct((NP, 128), jnp.float32),
    grid=(GRID,),
    in_specs=[_part_spec, _row_spec(DH), _deg_spec,
              _full_spec((DH, 128)), _full_spec((1, DH)), _full_spec((1, 128))],
    out_specs=_row_spec(128),
)


@jax.jit
def kernel(x, edge_index, W1, b1, W2, b2, Wc, bc):
    src = edge_index[0].astype(jnp.int32)
    dst = edge_index[1].astype(jnp.int32)
    pad = EP - src.shape[0]
    fill = jnp.full((pad,), N, dtype=jnp.int32)
    src_r = jnp.concatenate([src, fill]).reshape(NWORK, NCH, CHUNK)
    dst_r = jnp.concatenate([dst, fill]).reshape(NWORK, NCH, CHUNK)

    x_p = jnp.zeros((NP, DI), jnp.float32).at[:N].set(x)
    ones16 = jnp.ones((CHUNK, 16), jnp.float32)
    zeros16 = jnp.zeros((NP, 16), jnp.float32)
    zeros64 = jnp.zeros((NP, DH), jnp.float32)
    wc_p = jnp.zeros((DH, 128), jnp.float32).at[:, :2].set(Wc)
    bc_p = jnp.zeros((1, 128), jnp.float32).at[0, :2].set(bc)

    deg_kernel, agg_kernel = _sc_kernels()
    deg = deg_kernel(dst_r, ones16, zeros16)
    g1 = _stage_a(x_p, W1, deg)
    a1 = agg_kernel(g1, src_r, dst_r, zeros64)
    g2 = _stage_b(a1, g1, deg, W2, b1.reshape(1, DH))
    a2 = agg_kernel(g2, src_r, dst_r, zeros64)
    logits = _stage_c(a2, g2, deg, wc_p, b2.reshape(1, DH), bc_p)
    return logits[:N, :2]


# speedup vs baseline: 15.2167x; 1.1111x over previous
import functools

import jax
import jax.numpy as jnp
from jax import lax
from jax.experimental import pallas as pl
from jax.experimental.pallas import tpu as pltpu
from jax.experimental.pallas import tpu_sc as plsc

N = 10000
NP = 10240
DI = 128
DH = 64
E = 320000
NWORK = 32
CHUNK = 128
NCH = 80
EPW = NCH * CHUNK
EP = NWORK * EPW
SLAB = NP // 16

@functools.cache
def _sc_kernels():
    mesh = plsc.VectorSubcoreMesh(
        core_axis_name="c", subcore_axis_name="s", num_cores=2, num_subcores=16)

    @functools.partial(
        pl.kernel,
        out_type=jax.ShapeDtypeStruct((2, NP, 16), jnp.float32),
        mesh=mesh,
        compiler_params=pltpu.CompilerParams(use_tc_tiling_on_sc=False),
        scratch_types=[
            pltpu.VMEM((NCH, CHUNK), jnp.int32),
            pltpu.VMEM((CHUNK, 16), jnp.float32),
            pltpu.VMEM_SHARED((NP, 16), jnp.float32),
        ],
    )
    def deg_kernel(dst_hbm, ones_hbm, zeros_hbm, out_hbm, dst_v, ones_v, acc):
        cid = lax.axis_index("c")
        sid = lax.axis_index("s")
        wid = cid * 16 + sid
        slab = pl.ds(sid * SLAB, SLAB)
        pltpu.sync_copy(dst_hbm.at[wid], dst_v)
        pltpu.sync_copy(ones_hbm, ones_v)
        pltpu.sync_copy(zeros_hbm.at[slab], acc.at[slab])
        plsc.subcore_barrier()

        def body(j, carry):
            pltpu.sync_copy(ones_v, acc.at[dst_v.at[j]], add=True)
            return carry

        lax.fori_loop(0, NCH, body, 0)
        plsc.subcore_barrier()
        pltpu.sync_copy(acc.at[slab], out_hbm.at[cid].at[slab])

    @functools.partial(
        pl.kernel,
        out_type=jax.ShapeDtypeStruct((2, NP, DH), jnp.float32),
        mesh=mesh,
        compiler_params=pltpu.CompilerParams(use_tc_tiling_on_sc=False),
        scratch_types=[
            pltpu.VMEM((NCH, CHUNK), jnp.int32),
            pltpu.VMEM((NCH, CHUNK), jnp.int32),
            pltpu.VMEM((2, CHUNK, DH), jnp.float32),
            pltpu.VMEM_SHARED((NP, DH), jnp.float32),
            pltpu.SemaphoreType.DMA,
            pltpu.SemaphoreType.DMA,
        ],
    )
    def agg_kernel(g_hbm, src_hbm, dst_hbm, zeros_hbm, out_hbm,
                   src_v, dst_v, rows, acc, gsem0, gsem1):
        cid = lax.axis_index("c")
        sid = lax.axis_index("s")
        wid = cid * 16 + sid
        slab = pl.ds(sid * SLAB, SLAB)
        pltpu.sync_copy(src_hbm.at[wid], src_v)
        pltpu.sync_copy(dst_hbm.at[wid], dst_v)
        pltpu.sync_copy(zeros_hbm.at[slab], acc.at[slab])
        plsc.subcore_barrier()

        gsems = (gsem0, gsem1)

        def gather(j, b):
            return pltpu.async_copy(g_hbm.at[src_v.at[j]], rows.at[b],
                                    gsems[b])

        def gwait(b):
            pltpu.make_async_copy(g_hbm.at[src_v.at[0]], rows.at[b],
                                  gsems[b]).wait()

        def scatter(j, b):
            pltpu.sync_copy(rows.at[b], acc.at[dst_v.at[j]], add=True)

        gather(0, 0)

        def body(j2, carry):
            j = j2 * 2
            g1 = gather(j + 1, 1)
            gwait(0)
            scatter(j, 0)
            g1.wait()

            @pl.when(j2 < NCH // 2 - 1)
            def _():
                gather(j + 2, 0)

            scatter(j + 1, 1)
            return carry

        lax.fori_loop(0, NCH // 2, body, 0)
        plsc.subcore_barrier()
        pltpu.sync_copy(acc.at[slab], out_hbm.at[cid].at[slab])

    return deg_kernel, agg_kernel


RB = 512
GRID = NP // RB


def _dinv_from(deg_ref):
    d = deg_ref[0, :, 0:1] + deg_ref[1, :, 0:1] + 1.0
    return lax.rsqrt(d)


def _stage_a_body(x_ref, w_ref, deg_ref, g_ref):
    dinv = _dinv_from(deg_ref)
    h = jnp.dot(x_ref[...], w_ref[...], preferred_element_type=jnp.float32)
    g_ref[...] = h * dinv


def _stage_b_body(a_ref, g1_ref, deg_ref, w_ref, b_ref, g2_ref):
    dinv = _dinv_from(deg_ref)
    agg = a_ref[0] + a_ref[1] + g1_ref[...]
    z = jnp.maximum(agg * dinv + b_ref[...], 0.0)
    h2 = jnp.dot(z, w_ref[...], preferred_element_type=jnp.float32)
    g2_ref[...] = h2 * dinv


def _stage_c_body(a_ref, g2_ref, deg_ref, w_ref, b2_ref, bc_ref, out_ref):
    dinv = _dinv_from(deg_ref)
    agg = a_ref[0] + a_ref[1] + g2_ref[...]
    z = agg * dinv + b2_ref[...]
    out_ref[...] = (
        jnp.dot(z, w_ref[...], preferred_element_type=jnp.float32)
        + bc_ref[...])


def _row_spec(d):
    return pl.BlockSpec((RB, d), lambda i: (i, 0))


_deg_spec = pl.BlockSpec((2, RB, 16), lambda i: (0, i, 0))
_part_spec = pl.BlockSpec((2, RB, DH), lambda i: (0, i, 0))


def _full_spec(shape):
    return pl.BlockSpec(shape, lambda i: tuple(0 for _ in shape))


_stage_a = pl.pallas_call(
    _stage_a_body,
    out_shape=jax.ShapeDtypeStruct((NP, DH), jnp.float32),
    grid=(GRID,),
    in_specs=[_row_spec(DI), _full_spec((DI, DH)), _deg_spec],
    out_specs=_row_spec(DH),
)

_stage_b = pl.pallas_call(
    _stage_b_body,
    out_shape=jax.ShapeDtypeStruct((NP, DH), jnp.float32),
    grid=(GRID,),
    in_specs=[_part_spec, _row_spec(DH), _deg_spec,
              _full_spec((DH, DH)), _full_spec((1, DH))],
    out_specs=_row_spec(DH),
)

_stage_c = pl.pallas_call(
    _stage_c_body,
    out_shape=jax.ShapeDtypeStruct((NP, 128), jnp.float32),
    grid=(GRID,),
    in_specs=[_part_spec, _row_spec(DH), _deg_spec,
              _full_spec((DH, 128)), _full_spec((1, DH)), _full_spec((1, 128))],
    out_specs=_row_spec(128),
)


@jax.jit
def kernel(x, edge_index, W1, b1, W2, b2, Wc, bc):
    src = edge_index[0].astype(jnp.int32)
    dst = edge_index[1].astype(jnp.int32)
    pad = EP - src.shape[0]
    fill = jnp.full((pad,), N, dtype=jnp.int32)
    src_r = jnp.concatenate([src, fill]).reshape(NWORK, NCH, CHUNK)
    dst_r = jnp.concatenate([dst, fill]).reshape(NWORK, NCH, CHUNK)

    x_p = jnp.zeros((NP, DI), jnp.float32).at[:N].set(x)
    ones16 = jnp.ones((CHUNK, 16), jnp.float32)
    zeros16 = jnp.zeros((NP, 16), jnp.float32)
    zeros64 = jnp.zeros((NP, DH), jnp.float32)
    wc_p = jnp.zeros((DH, 128), jnp.float32).at[:, :2].set(Wc)
    bc_p = jnp.zeros((1, 128), jnp.float32).at[0, :2].set(bc)

    deg_kernel, agg_kernel = _sc_kernels()
    deg = deg_kernel(dst_r, ones16, zeros16)
    g1 = _stage_a(x_p, W1, deg)
    a1 = agg_kernel(g1, src_r, dst_r, zeros64)
    g2 = _stage_b(a1, g1, deg, W2, b1.reshape(1, DH))
    a2 = agg_kernel(g2, src_r, dst_r, zeros64)
    logits = _stage_c(a2, g2, deg, wc_p, b2.reshape(1, DH), bc_p)
    return logits[:N, :2]


# speedup vs baseline: 31.5472x; 2.0732x over previous
import functools

import jax
import jax.numpy as jnp
from jax import lax
from jax.experimental import pallas as pl
from jax.experimental.pallas import tpu as pltpu
from jax.experimental.pallas import tpu_sc as plsc

N = 10000
NP = 10240
DI = 128
DH = 64
E = 320000
NWORK = 32
CHUNK = 128
NCH = 80
EPW = NCH * CHUNK
EP = NWORK * EPW
SLAB = NP // 16

@functools.cache
def _sc_kernels():
    mesh = plsc.VectorSubcoreMesh(
        core_axis_name="c", subcore_axis_name="s", num_cores=2, num_subcores=16)

    @functools.partial(
        pl.kernel,
        out_type=jax.ShapeDtypeStruct((2, NP, 16), jnp.float32),
        mesh=mesh,
        compiler_params=pltpu.CompilerParams(use_tc_tiling_on_sc=False),
        scratch_types=[
            pltpu.VMEM((NCH, CHUNK), jnp.int32),
            pltpu.VMEM((CHUNK, 16), jnp.float32),
            pltpu.VMEM_SHARED((NP, 16), jnp.float32),
        ],
    )
    def deg_kernel(dst_hbm, ones_hbm, zeros_hbm, out_hbm, dst_v, ones_v, acc):
        cid = lax.axis_index("c")
        sid = lax.axis_index("s")
        wid = cid * 16 + sid
        slab = pl.ds(sid * SLAB, SLAB)
        pltpu.sync_copy(dst_hbm.at[wid], dst_v)
        pltpu.sync_copy(ones_hbm, ones_v)
        pltpu.sync_copy(zeros_hbm.at[slab], acc.at[slab])
        plsc.subcore_barrier()

        def body(j, carry):
            pltpu.sync_copy(ones_v, acc.at[dst_v.at[j]], add=True)
            return carry

        lax.fori_loop(0, NCH, body, 0)
        plsc.subcore_barrier()
        pltpu.sync_copy(acc.at[slab], out_hbm.at[cid].at[slab])

    @functools.partial(
        pl.kernel,
        out_type=jax.ShapeDtypeStruct((2, NP, DH), jnp.float32),
        mesh=mesh,
        compiler_params=pltpu.CompilerParams(use_tc_tiling_on_sc=False),
        scratch_types=[
            pltpu.VMEM((NCH, CHUNK), jnp.int32),
            pltpu.VMEM((NCH, CHUNK), jnp.int32),
            pltpu.VMEM((2, CHUNK, DH), jnp.float32),
            pltpu.VMEM_SHARED((NP, DH), jnp.float32),
            pltpu.VMEM_SHARED((NP, DH), jnp.float32),
            pltpu.SemaphoreType.DMA,
            pltpu.SemaphoreType.DMA,
        ],
    )
    def agg_kernel(g_hbm, src_hbm, dst_hbm, zeros_hbm, out_hbm,
                   src_v, dst_v, rows, acc, g_sh, gsem0, gsem1):
        cid = lax.axis_index("c")
        sid = lax.axis_index("s")
        wid = cid * 16 + sid
        slab = pl.ds(sid * SLAB, SLAB)
        pltpu.sync_copy(src_hbm.at[wid], src_v)
        pltpu.sync_copy(dst_hbm.at[wid], dst_v)
        pltpu.sync_copy(zeros_hbm.at[slab], acc.at[slab])
        pltpu.sync_copy(g_hbm.at[slab], g_sh.at[slab])
        plsc.subcore_barrier()

        gsems = (gsem0, gsem1)

        def gather(j, b):
            return pltpu.async_copy(g_sh.at[src_v.at[j]], rows.at[b],
                                    gsems[b])

        def gwait(b):
            pltpu.make_async_copy(g_sh.at[src_v.at[0]], rows.at[b],
                                  gsems[b]).wait()

        def scatter(j, b):
            pltpu.sync_copy(rows.at[b], acc.at[dst_v.at[j]], add=True)

        gather(0, 0)

        def body(j2, carry):
            j = j2 * 2
            g1 = gather(j + 1, 1)
            gwait(0)
            scatter(j, 0)
            g1.wait()

            @pl.when(j2 < NCH // 2 - 1)
            def _():
                gather(j + 2, 0)

            scatter(j + 1, 1)
            return carry

        lax.fori_loop(0, NCH // 2, body, 0)
        plsc.subcore_barrier()
        pltpu.sync_copy(acc.at[slab], out_hbm.at[cid].at[slab])

    return deg_kernel, agg_kernel


RB = 512
GRID = NP // RB


def _dinv_from(deg_ref):
    d = deg_ref[0, :, 0:1] + deg_ref[1, :, 0:1] + 1.0
    return lax.rsqrt(d)


def _stage_a_body(x_ref, w_ref, deg_ref, g_ref):
    dinv = _dinv_from(deg_ref)
    h = jnp.dot(x_ref[...], w_ref[...], preferred_element_type=jnp.float32)
    g_ref[...] = h * dinv


def _stage_b_body(a_ref, g1_ref, deg_ref, w_ref, b_ref, g2_ref):
    dinv = _dinv_from(deg_ref)
    agg = a_ref[0] + a_ref[1] + g1_ref[...]
    z = jnp.maximum(agg * dinv + b_ref[...], 0.0)
    h2 = jnp.dot(z, w_ref[...], preferred_element_type=jnp.float32)
    g2_ref[...] = h2 * dinv


def _stage_c_body(a_ref, g2_ref, deg_ref, w_ref, b2_ref, bc_ref, out_ref):
    dinv = _dinv_from(deg_ref)
    agg = a_ref[0] + a_ref[1] + g2_ref[...]
    z = agg * dinv + b2_ref[...]
    out_ref[...] = (
        jnp.dot(z, w_ref[...], preferred_element_type=jnp.float32)
        + bc_ref[...])


def _row_spec(d):
    return pl.BlockSpec((RB, d), lambda i: (i, 0))


_deg_spec = pl.BlockSpec((2, RB, 16), lambda i: (0, i, 0))
_part_spec = pl.BlockSpec((2, RB, DH), lambda i: (0, i, 0))


def _full_spec(shape):
    return pl.BlockSpec(shape, lambda i: tuple(0 for _ in shape))


_stage_a = pl.pallas_call(
    _stage_a_body,
    out_shape=jax.ShapeDtypeStruct((NP, DH), jnp.float32),
    grid=(GRID,),
    in_specs=[_row_spec(DI), _full_spec((DI, DH)), _deg_spec],
    out_specs=_row_spec(DH),
)

_stage_b = pl.pallas_call(
    _stage_b_body,
    out_shape=jax.ShapeDtypeStruct((NP, DH), jnp.float32),
    grid=(GRID,),
    in_specs=[_part_spec, _row_spec(DH), _deg_spec,
              _full_spec((DH, DH)), _full_spec((1, DH))],
    out_specs=_row_spec(DH),
)

_stage_c = pl.pallas_call(
    _stage_c_body,
    out_shape=jax.ShapeDtypeStruct((NP, 128), jnp.float32),
    grid=(GRID,),
    in_specs=[_part_spec, _row_spec(DH), _deg_spec,
              _full_spec((DH, 128)), _full_spec((1, DH)), _full_spec((1, 128))],
    out_specs=_row_spec(128),
)


@jax.jit
def kernel(x, edge_index, W1, b1, W2, b2, Wc, bc):
    src = edge_index[0].astype(jnp.int32)
    dst = edge_index[1].astype(jnp.int32)
    pad = EP - src.shape[0]
    fill = jnp.full((pad,), N, dtype=jnp.int32)
    src_r = jnp.concatenate([src, fill]).reshape(NWORK, NCH, CHUNK)
    dst_r = jnp.concatenate([dst, fill]).reshape(NWORK, NCH, CHUNK)

    x_p = jnp.zeros((NP, DI), jnp.float32).at[:N].set(x)
    ones16 = jnp.ones((CHUNK, 16), jnp.float32)
    zeros16 = jnp.zeros((NP, 16), jnp.float32)
    zeros64 = jnp.zeros((NP, DH), jnp.float32)
    wc_p = jnp.zeros((DH, 128), jnp.float32).at[:, :2].set(Wc)
    bc_p = jnp.zeros((1, 128), jnp.float32).at[0, :2].set(bc)

    deg_kernel, agg_kernel = _sc_kernels()
    deg = deg_kernel(dst_r, ones16, zeros16)
    g1 = _stage_a(x_p, W1, deg)
    a1 = agg_kernel(g1, src_r, dst_r, zeros64)
    g2 = _stage_b(a1, g1, deg, W2, b1.reshape(1, DH))
    a2 = agg_kernel(g2, src_r, dst_r, zeros64)
    logits = _stage_c(a2, g2, deg, wc_p, b2.reshape(1, DH), bc_p)
    return logits[:N, :2]


# speedup vs baseline: 34.0104x; 1.0781x over previous
import functools

import jax
import jax.numpy as jnp
from jax import lax
from jax.experimental import pallas as pl
from jax.experimental.pallas import tpu as pltpu
from jax.experimental.pallas import tpu_sc as plsc

N = 10000
NP = 10240
DI = 128
DH = 64
E = 320000
NWORK = 32
CHUNK = 128
NCH = 80
EPW = NCH * CHUNK
EP = NWORK * EPW
SLAB = NP // 16

@functools.cache
def _sc_kernels():
    mesh = plsc.VectorSubcoreMesh(
        core_axis_name="c", subcore_axis_name="s", num_cores=2, num_subcores=16)

    @functools.partial(
        pl.kernel,
        out_type=jax.ShapeDtypeStruct((2, NP), jnp.float32),
        mesh=mesh,
        compiler_params=pltpu.CompilerParams(use_tc_tiling_on_sc=False),
        scratch_types=[
            pltpu.VMEM((NCH, CHUNK), jnp.int32),
            pltpu.VMEM((CHUNK,), jnp.float32),
            pltpu.VMEM_SHARED((NP,), jnp.float32),
        ],
    )
    def deg_kernel(dst_hbm, ones_hbm, zeros_hbm, out_hbm, dst_v, ones_v, acc):
        cid = lax.axis_index("c")
        sid = lax.axis_index("s")
        wid = cid * 16 + sid
        slab = pl.ds(sid * SLAB, SLAB)
        pltpu.sync_copy(dst_hbm.at[wid], dst_v)
        pltpu.sync_copy(ones_hbm, ones_v)
        pltpu.sync_copy(zeros_hbm.at[slab], acc.at[slab])
        plsc.subcore_barrier()

        def body(j, carry):
            pltpu.sync_copy(ones_v, acc.at[dst_v.at[j]], add=True)
            return carry

        lax.fori_loop(0, NCH, body, 0)
        plsc.subcore_barrier()
        pltpu.sync_copy(acc.at[slab], out_hbm.at[cid].at[slab])

    @functools.partial(
        pl.kernel,
        out_type=jax.ShapeDtypeStruct((2, NP, DH), jnp.float32),
        mesh=mesh,
        compiler_params=pltpu.CompilerParams(use_tc_tiling_on_sc=False),
        scratch_types=[
            pltpu.VMEM((NCH, CHUNK), jnp.int32),
            pltpu.VMEM((NCH, CHUNK), jnp.int32),
            pltpu.VMEM((2, CHUNK, DH), jnp.float32),
            pltpu.VMEM_SHARED((NP, DH), jnp.float32),
            pltpu.VMEM_SHARED((NP, DH), jnp.float32),
            pltpu.SemaphoreType.DMA,
            pltpu.SemaphoreType.DMA,
        ],
    )
    def agg_kernel(g_hbm, src_hbm, dst_hbm, zeros_hbm, out_hbm,
                   src_v, dst_v, rows, acc, g_sh, gsem0, gsem1):
        cid = lax.axis_index("c")
        sid = lax.axis_index("s")
        wid = cid * 16 + sid
        slab = pl.ds(sid * SLAB, SLAB)
        pltpu.sync_copy(src_hbm.at[wid], src_v)
        pltpu.sync_copy(dst_hbm.at[wid], dst_v)
        pltpu.sync_copy(zeros_hbm.at[slab], acc.at[slab])
        pltpu.sync_copy(g_hbm.at[slab], g_sh.at[slab])
        plsc.subcore_barrier()

        gsems = (gsem0, gsem1)

        def gather(j, b):
            return pltpu.async_copy(g_sh.at[src_v.at[j]], rows.at[b],
                                    gsems[b])

        def gwait(b):
            pltpu.make_async_copy(g_sh.at[src_v.at[0]], rows.at[b],
                                  gsems[b]).wait()

        def scatter(j, b):
            pltpu.sync_copy(rows.at[b], acc.at[dst_v.at[j]], add=True)

        gather(0, 0)

        def body(j2, carry):
            j = j2 * 2
            g1 = gather(j + 1, 1)
            gwait(0)
            scatter(j, 0)
            g1.wait()

            @pl.when(j2 < NCH // 2 - 1)
            def _():
                gather(j + 2, 0)

            scatter(j + 1, 1)
            return carry

        lax.fori_loop(0, NCH // 2, body, 0)
        plsc.subcore_barrier()
        pltpu.sync_copy(acc.at[slab], out_hbm.at[cid].at[slab])

    return deg_kernel, agg_kernel


RB = 512
GRID = NP // RB


def _stage_a_body(x_ref, w_ref, h_ref):
    h_ref[...] = jnp.dot(x_ref[...], w_ref[...],
                         preferred_element_type=jnp.float32)


def _stage_b_body(a_ref, g1_ref, dinv_ref, w_ref, b_ref, g2_ref):
    dinv = dinv_ref[...]
    agg = a_ref[0] + a_ref[1] + g1_ref[...]
    z = jnp.maximum(agg * dinv + b_ref[...], 0.0)
    h2 = jnp.dot(z, w_ref[...], preferred_element_type=jnp.float32)
    g2_ref[...] = h2 * dinv


def _stage_c_body(a_ref, g2_ref, dinv_ref, w_ref, b2_ref, bc_ref, out_ref):
    agg = a_ref[0] + a_ref[1] + g2_ref[...]
    z = agg * dinv_ref[...] + b2_ref[...]
    out_ref[...] = (
        jnp.dot(z, w_ref[...], preferred_element_type=jnp.float32)
        + bc_ref[...])


def _row_spec(d):
    return pl.BlockSpec((RB, d), lambda i: (i, 0))


_part_spec = pl.BlockSpec((2, RB, DH), lambda i: (0, i, 0))


def _full_spec(shape):
    return pl.BlockSpec(shape, lambda i: tuple(0 for _ in shape))


_stage_a = pl.pallas_call(
    _stage_a_body,
    out_shape=jax.ShapeDtypeStruct((NP, DH), jnp.float32),
    grid=(GRID,),
    in_specs=[_row_spec(DI), _full_spec((DI, DH))],
    out_specs=_row_spec(DH),
)

_stage_b = pl.pallas_call(
    _stage_b_body,
    out_shape=jax.ShapeDtypeStruct((NP, DH), jnp.float32),
    grid=(GRID,),
    in_specs=[_part_spec, _row_spec(DH), _row_spec(1),
              _full_spec((DH, DH)), _full_spec((1, DH))],
    out_specs=_row_spec(DH),
)

_stage_c = pl.pallas_call(
    _stage_c_body,
    out_shape=jax.ShapeDtypeStruct((NP, 128), jnp.float32),
    grid=(GRID,),
    in_specs=[_part_spec, _row_spec(DH), _row_spec(1),
              _full_spec((DH, 128)), _full_spec((1, DH)), _full_spec((1, 128))],
    out_specs=_row_spec(128),
)


@jax.jit
def kernel(x, edge_index, W1, b1, W2, b2, Wc, bc):
    src = edge_index[0].astype(jnp.int32)
    dst = edge_index[1].astype(jnp.int32)
    pad = EP - src.shape[0]
    fill = jnp.full((pad,), N, dtype=jnp.int32)
    src_r = jnp.concatenate([src, fill]).reshape(NWORK, NCH, CHUNK)
    dst_r = jnp.concatenate([dst, fill]).reshape(NWORK, NCH, CHUNK)

    x_p = jnp.zeros((NP, DI), jnp.float32).at[:N].set(x)
    ones1 = jnp.ones((CHUNK,), jnp.float32)
    zeros1 = jnp.zeros((NP,), jnp.float32)
    zeros64 = jnp.zeros((NP, DH), jnp.float32)
    wc_p = jnp.zeros((DH, 128), jnp.float32).at[:, :2].set(Wc)
    bc_p = jnp.zeros((1, 128), jnp.float32).at[0, :2].set(bc)

    deg_kernel, agg_kernel = _sc_kernels()
    degp = deg_kernel(dst_r, ones1, zeros1)
    h1 = _stage_a(x_p, W1)
    dinv = lax.rsqrt(degp[0] + degp[1] + 1.0)[:, None]
    g1 = h1 * dinv
    a1 = agg_kernel(g1, src_r, dst_r, zeros64)
    g2 = _stage_b(a1, g1, dinv, W2, b1.reshape(1, DH))
    a2 = agg_kernel(g2, src_r, dst_r, zeros64)
    logits = _stage_c(a2, g2, dinv, wc_p, b2.reshape(1, DH), bc_p)
    return logits[:N, :2]


# speedup vs baseline: 39.0820x; 1.1491x over previous
import functools

import jax
import jax.numpy as jnp
from jax import lax
from jax.experimental import pallas as pl
from jax.experimental.pallas import tpu as pltpu
from jax.experimental.pallas import tpu_sc as plsc

N = 10000
DI = 128
DH = 64
E = 320000
NWORK = 32
CHUNK = 128
NROW = E // CHUNK
NCH = NROW // NWORK
NEXTRA = NROW - NCH * NWORK
SLAB = N // 16


@functools.cache
def _sc_kernels():
    mesh = plsc.VectorSubcoreMesh(
        core_axis_name="c", subcore_axis_name="s", num_cores=2, num_subcores=16)

    @functools.partial(
        pl.kernel,
        out_type=jax.ShapeDtypeStruct((2, N), jnp.float32),
        mesh=mesh,
        compiler_params=pltpu.CompilerParams(use_tc_tiling_on_sc=False),
        scratch_types=[
            pltpu.VMEM((NCH + 1, CHUNK), jnp.int32),
            pltpu.VMEM((CHUNK,), jnp.float32),
            pltpu.VMEM_SHARED((N,), jnp.float32),
        ],
    )
    def deg_kernel(edges_hbm, ones_hbm, zeros_hbm, out_hbm, dst_v, ones_v, acc):
        cid = lax.axis_index("c")
        sid = lax.axis_index("s")
        wid = cid * 16 + sid
        dst_hbm = edges_hbm.at[1]
        pltpu.sync_copy(dst_hbm.at[pl.ds(wid * NCH, NCH)],
                        dst_v.at[pl.ds(0, NCH)])
        pltpu.sync_copy(ones_hbm, ones_v)
        deg_slab = pl.ds(sid * 1000, 1000)

        @pl.when(sid < 10)
        def _():
            pltpu.sync_copy(zeros_hbm.at[deg_slab], acc.at[deg_slab])

        @pl.when(wid < NEXTRA)
        def _():
            pltpu.sync_copy(dst_hbm.at[pl.ds(NCH * NWORK + wid, 1)],
                            dst_v.at[pl.ds(NCH, 1)])

        plsc.subcore_barrier()

        def body(j, carry):
            pltpu.sync_copy(ones_v, acc.at[dst_v.at[j]], add=True)
            return carry

        lax.fori_loop(0, NCH, body, 0)

        @pl.when(wid < NEXTRA)
        def _():
            pltpu.sync_copy(ones_v, acc.at[dst_v.at[NCH]], add=True)

        plsc.subcore_barrier()

        @pl.when(sid < 10)
        def _():
            pltpu.sync_copy(acc.at[deg_slab], out_hbm.at[cid].at[deg_slab])

    @functools.partial(
        pl.kernel,
        out_type=jax.ShapeDtypeStruct((2, N, DH), jnp.float32),
        mesh=mesh,
        compiler_params=pltpu.CompilerParams(use_tc_tiling_on_sc=False),
        scratch_types=[
            pltpu.VMEM((NCH + 1, CHUNK), jnp.int32),
            pltpu.VMEM((NCH + 1, CHUNK), jnp.int32),
            pltpu.VMEM((2, CHUNK, DH), jnp.float32),
            pltpu.VMEM_SHARED((N, DH), jnp.float32),
            pltpu.VMEM_SHARED((N, DH), jnp.float32),
            pltpu.SemaphoreType.DMA,
            pltpu.SemaphoreType.DMA,
        ],
    )
    def agg_kernel(g_hbm, edges_hbm, zeros_hbm, out_hbm,
                   src_v, dst_v, rows, acc, g_sh, gsem0, gsem1):
        cid = lax.axis_index("c")
        sid = lax.axis_index("s")
        wid = cid * 16 + sid
        slab = pl.ds(sid * SLAB, SLAB)
        pltpu.sync_copy(edges_hbm.at[0].at[pl.ds(wid * NCH, NCH)],
                        src_v.at[pl.ds(0, NCH)])
        pltpu.sync_copy(edges_hbm.at[1].at[pl.ds(wid * NCH, NCH)],
                        dst_v.at[pl.ds(0, NCH)])

        @pl.when(wid < NEXTRA)
        def _():
            pltpu.sync_copy(edges_hbm.at[0].at[pl.ds(NCH * NWORK + wid, 1)],
                            src_v.at[pl.ds(NCH, 1)])
            pltpu.sync_copy(edges_hbm.at[1].at[pl.ds(NCH * NWORK + wid, 1)],
                            dst_v.at[pl.ds(NCH, 1)])

        pltpu.sync_copy(zeros_hbm.at[slab], acc.at[slab])
        pltpu.sync_copy(g_hbm.at[slab], g_sh.at[slab])
        plsc.subcore_barrier()

        gsems = (gsem0, gsem1)

        def gather(j, b):
            return pltpu.async_copy(g_sh.at[src_v.at[j]], rows.at[b],
                                    gsems[b])

        def gwait(b):
            pltpu.make_async_copy(g_sh.at[src_v.at[0]], rows.at[b],
                                  gsems[b]).wait()

        def scatter(j, b):
            pltpu.sync_copy(rows.at[b], acc.at[dst_v.at[j]], add=True)

        gather(0, 0)

        def body(j2, carry):
            j = j2 * 2
            g1 = gather(j + 1, 1)
            gwait(0)
            scatter(j, 0)
            g1.wait()

            @pl.when(j2 < NCH // 2 - 1)
            def _():
                gather(j + 2, 0)

            scatter(j + 1, 1)
            return carry

        lax.fori_loop(0, NCH // 2, body, 0)

        @pl.when(wid < NEXTRA)
        def _():
            pltpu.async_copy(g_sh.at[src_v.at[NCH]], rows.at[0],
                             gsems[0]).wait()
            scatter(NCH, 0)

        plsc.subcore_barrier()
        pltpu.sync_copy(acc.at[slab], out_hbm.at[cid].at[slab])

    return deg_kernel, agg_kernel


RB = 2000
GRID = N // RB


def _stage_a_body(x_ref, w_ref, h_ref):
    h_ref[...] = jnp.dot(x_ref[...], w_ref[...],
                         preferred_element_type=jnp.float32)


def _stage_b_body(a_ref, g1_ref, dinv_ref, w_ref, b_ref, g2_ref):
    dinv = dinv_ref[...]
    agg = a_ref[0] + a_ref[1] + g1_ref[...]
    z = jnp.maximum(agg * dinv + b_ref[...], 0.0)
    h2 = jnp.dot(z, w_ref[...], preferred_element_type=jnp.float32)
    g2_ref[...] = h2 * dinv


def _stage_c_body(a_ref, g2_ref, dinv_ref, w_ref, b2_ref, bc_ref, out_ref):
    agg = a_ref[0] + a_ref[1] + g2_ref[...]
    z = agg * dinv_ref[...] + b2_ref[...]
    out_ref[...] = (
        jnp.dot(z, w_ref[...], preferred_element_type=jnp.float32)
        + bc_ref[...])


def _row_spec(d):
    return pl.BlockSpec((RB, d), lambda i: (i, 0))


_part_spec = pl.BlockSpec((2, RB, DH), lambda i: (0, i, 0))


def _full_spec(shape):
    return pl.BlockSpec(shape, lambda i: tuple(0 for _ in shape))


_stage_a = pl.pallas_call(
    _stage_a_body,
    out_shape=jax.ShapeDtypeStruct((N, DH), jnp.float32),
    grid=(GRID,),
    in_specs=[_row_spec(DI), _full_spec((DI, DH))],
    out_specs=_row_spec(DH),
)

_stage_b = pl.pallas_call(
    _stage_b_body,
    out_shape=jax.ShapeDtypeStruct((N, DH), jnp.float32),
    grid=(GRID,),
    in_specs=[_part_spec, _row_spec(DH), _row_spec(1),
              _full_spec((DH, DH)), _full_spec((1, DH))],
    out_specs=_row_spec(DH),
)

_stage_c = pl.pallas_call(
    _stage_c_body,
    out_shape=jax.ShapeDtypeStruct((N, 8), jnp.float32),
    grid=(GRID,),
    in_specs=[_part_spec, _row_spec(DH), _row_spec(1),
              _full_spec((DH, 8)), _full_spec((1, DH)), _full_spec((1, 8))],
    out_specs=_row_spec(8),
)


@jax.jit
def kernel(x, edge_index, W1, b1, W2, b2, Wc, bc):
    edges = edge_index.astype(jnp.int32).reshape(2, NROW, CHUNK)

    ones1 = jnp.ones((CHUNK,), jnp.float32)
    zeros1 = jnp.zeros((N,), jnp.float32)
    zeros64 = jnp.zeros((N, DH), jnp.float32)
    wc_p = jnp.zeros((DH, 8), jnp.float32).at[:, :2].set(Wc)
    bc_p = jnp.zeros((1, 8), jnp.float32).at[0, :2].set(bc)

    deg_kernel, agg_kernel = _sc_kernels()
    degp = deg_kernel(edges, ones1, zeros1)
    h1 = _stage_a(x, W1)
    dinv = lax.rsqrt(degp[0] + degp[1] + 1.0)[:, None]
    g1 = h1 * dinv
    a1 = agg_kernel(g1, edges, zeros64)
    g2 = _stage_b(a1, g1, dinv, W2, b1.reshape(1, DH))
    a2 = agg_kernel(g2, edges, zeros64)
    logits = _stage_c(a2, g2, dinv, wc_p, b2.reshape(1, DH), bc_p)
    return logits[:, :2]
